# Initial kernel scaffold; baseline (speedup 1.0000x reference)
import jax
import jax.numpy as jnp
from jax.experimental import pallas as pl


def kernel(x, edge_index, edge_weight, W, bias, gamma, beta):
    raise NotImplementedError("write your pallas kernel here")


# speedup vs baseline: 2.8390x; 2.8390x over previous
import functools

import jax
import jax.numpy as jnp
from jax import lax
from jax.experimental import pallas as pl
from jax.experimental.pallas import tpu as pltpu
from jax.experimental.pallas import tpu_sc as plsc

N = 10000
E = 160000
D = 256
DH = D // 2
C = float(D)
MIN_NORM = 1e-15
SQRT_C = C ** 0.5
MAXNORM = (1.0 - 4e-3) / SQRT_C

NS = 16
EB = 80
EPT = E // NS
NBATCH = EPT // EB
RQ = 624


def _make_spmm():
    mesh = plsc.VectorSubcoreMesh(
        core_axis_name="c", subcore_axis_name="s", num_cores=2)

    @functools.partial(
        pl.kernel,
        out_type=jax.ShapeDtypeStruct((2 * N, DH), jnp.float32),
        mesh=mesh,
        scratch_types=[
            pltpu.VMEM((EB,), jnp.int32),
            pltpu.VMEM((EB,), jnp.int32),
            pltpu.VMEM((EB,), jnp.float32),
            pltpu.VMEM((EB, DH), jnp.float32),
            pltpu.VMEM_SHARED((N, DH), jnp.float32),
            pltpu.SemaphoreType.DMA,
        ],
    )
    def spmm(x_hbm, col_hbm, row_hbm, w_hbm, z_hbm, out_hbm,
             gidx_v, ridx_v, w_v, rows_v, ysp, sem):
        c = lax.axis_index("c")
        s = lax.axis_index("s")
        cN = c * N

        pltpu.sync_copy(z_hbm, ysp.at[pl.ds(s * RQ, RQ)])

        @pl.when(s == NS - 1)
        def _():
            pltpu.sync_copy(z_hbm.at[pl.ds(0, 16)],
                            ysp.at[pl.ds(NS * RQ, 16)])

        plsc.subcore_barrier()

        base0 = s * EPT

        def batch_body(b, _):
            base = base0 + b * EB
            pltpu.sync_copy(col_hbm.at[pl.ds(base, EB)], gidx_v)
            pltpu.sync_copy(row_hbm.at[pl.ds(base, EB)], ridx_v)
            pltpu.sync_copy(w_hbm.at[pl.ds(base, EB)], w_v)

            def shift_body(i, _):
                sl = pl.ds(i * 16, 16)
                gidx_v[sl] = gidx_v[sl] + cN
                return 0
            lax.fori_loop(0, EB // 16, shift_body, 0, unroll=True)

            pltpu.async_copy(x_hbm.at[gidx_v], rows_v, sem).wait()

            def scale_body(g, _):
                wg = w_v[pl.ds(g * 16, 16)]
                for j in range(16):
                    we = wg[j]
                    e = g * 16 + j
                    for k in range(DH // 16):
                        sl = pl.ds(k * 16, 16)
                        rows_v[e, sl] = rows_v[e, sl] * we
                return 0
            lax.fori_loop(0, EB // 16, scale_body, 0)

            pltpu.sync_copy(rows_v, ysp.at[ridx_v], add=True)
            return 0

        lax.fori_loop(0, NBATCH, batch_body, 0)
        plsc.subcore_barrier()

        pltpu.sync_copy(ysp.at[pl.ds(s * RQ, RQ)],
                        out_hbm.at[pl.ds(cN + s * RQ, RQ)])

        @pl.when(s == NS - 1)
        def _():
            pltpu.sync_copy(ysp.at[pl.ds(NS * RQ, 16)],
                            out_hbm.at[pl.ds(cN + NS * RQ, 16)])

    return spmm


_spmm_cache = []


def _get_spmm():
    if not _spmm_cache:
        _spmm_cache.append(_make_spmm())
    return _spmm_cache[0]


BR = 1000
NBR = N // BR


def _rownorm(x):
    return jnp.sqrt(jnp.clip(jnp.sum(x * x, axis=-1, keepdims=True),
                             MIN_NORM * MIN_NORM, None))


def _clipnorm(n):
    return jnp.clip(n, MIN_NORM, None)


def _s1_body(y0_ref, y1_ref, w_ref, bias_ref, h_ref, acc_ref):
    i = pl.program_id(0)
    w = w_ref[...]
    h = lax.dot_general(y0_ref[...], w[:, :DH], (((1,), (1,)), ((), ())),
                        preferred_element_type=jnp.float32)
    h = h + lax.dot_general(y1_ref[...], w[:, DH:], (((1,), (1,)), ((), ())),
                            preferred_element_type=jnp.float32)

    un = _clipnorm(_rownorm(h))
    e = jnp.tanh(SQRT_C * un) * h / (SQRT_C * un)
    ne = _clipnorm(_rownorm(e))
    e = jnp.where(ne > MAXNORM, e / ne * MAXNORM, e)

    b = bias_ref[0, 0]
    bn = jnp.clip(jnp.abs(b), MIN_NORM, None)
    eb = jnp.tanh(SQRT_C * bn) * b / (SQRT_C * bn)
    nb = jnp.clip(jnp.abs(eb), MIN_NORM, None)
    vb = jnp.where(nb > MAXNORM, eb / nb * MAXNORM, eb)

    x2 = jnp.sum(e * e, axis=-1, keepdims=True)
    y2 = vb * vb
    xy = vb * jnp.sum(e, axis=-1, keepdims=True)
    num = (1.0 + 2.0 * C * xy + C * y2) * e + (1.0 - C * x2) * vb
    den = 1.0 + 2.0 * C * xy + C * C * x2 * y2
    m = num / jnp.clip(den, MIN_NORM, None)

    nm = _clipnorm(_rownorm(m))
    r = jnp.where(nm > MAXNORM, m / nm * MAXNORM, m)
    pn = _clipnorm(_rownorm(r))
    sarg = jnp.clip(SQRT_C * pn, -1.0 + 1e-7, 1.0 - 1e-7)
    atanh = 0.5 * jnp.log((1.0 + sarg) / (1.0 - sarg))
    hl = atanh * r / (SQRT_C * pn)

    h_ref[...] = hl

    @pl.when(i == 0)
    def _():
        acc_ref[...] = jnp.zeros_like(acc_ref)

    ssum = jnp.sum(hl, axis=0, keepdims=True)
    ssq = jnp.sum(hl * hl, axis=0, keepdims=True)
    upd = jnp.concatenate(
        [ssum, ssq, jnp.zeros((6, D), jnp.float32)], axis=0)
    acc_ref[...] = acc_ref[...] + upd


def _stage1(y01, W, bias2d):
    return pl.pallas_call(
        _s1_body,
        grid=(NBR,),
        in_specs=[
            pl.BlockSpec((BR, DH), lambda i: (i, 0)),
            pl.BlockSpec((BR, DH), lambda i: (NBR + i, 0)),
            pl.BlockSpec((D, D), lambda i: (0, 0)),
            pl.BlockSpec((1, 1), lambda i: (0, 0)),
        ],
        out_specs=[
            pl.BlockSpec((BR, D), lambda i: (i, 0)),
            pl.BlockSpec((8, D), lambda i: (0, 0)),
        ],
        out_shape=[
            jax.ShapeDtypeStruct((N, D), jnp.float32),
            jax.ShapeDtypeStruct((8, D), jnp.float32),
        ],
        compiler_params=pltpu.CompilerParams(
            dimension_semantics=("arbitrary",)),
    )(y01, y01, W, bias2d)


def _s3_body(h_ref, acc_ref, gamma_ref, beta_ref, out_ref):
    h = h_ref[...]
    mean = acc_ref[0:1, :] * (1.0 / N)
    ex2 = acc_ref[1:2, :] * (1.0 / N)
    var = ex2 - mean * mean
    xn = (h - mean) / jnp.sqrt(var + 1e-5) * gamma_ref[...] + beta_ref[...]
    out_ref[...] = h + jnp.maximum(xn, 0.0)


def _stage3(h, acc, gamma2d, beta2d):
    return pl.pallas_call(
        _s3_body,
        grid=(NBR,),
        in_specs=[
            pl.BlockSpec((BR, D), lambda i: (i, 0)),
            pl.BlockSpec((8, D), lambda i: (0, 0)),
            pl.BlockSpec((1, D), lambda i: (0, 0)),
            pl.BlockSpec((1, D), lambda i: (0, 0)),
        ],
        out_specs=pl.BlockSpec((BR, D), lambda i: (i, 0)),
        out_shape=jax.ShapeDtypeStruct((N, D), jnp.float32),
        compiler_params=pltpu.CompilerParams(
            dimension_semantics=("arbitrary",)),
    )(h, acc, gamma2d, beta2d)


def kernel(x, edge_index, edge_weight, W, bias, gamma, beta):
    row = edge_index[0]
    col = edge_index[1]
    x01 = jnp.concatenate([x[:, :DH], x[:, DH:]], axis=0)
    zrows = jnp.zeros((RQ, DH), jnp.float32)
    y01 = _get_spmm()(x01, col, row, edge_weight, zrows)
    h, acc = _stage1(y01, W, bias.reshape(1, 1))
    out = _stage3(h, acc, gamma.reshape(1, D), beta.reshape(1, D))
    return out


# speedup vs baseline: 6.8573x; 2.4154x over previous
import functools

import jax
import jax.numpy as jnp
from jax import lax
from jax.experimental import pallas as pl
from jax.experimental.pallas import tpu as pltpu
from jax.experimental.pallas import tpu_sc as plsc

N = 10000
E = 160000
D = 256
DH = D // 2
C = float(D)
MIN_NORM = 1e-15
SQRT_C = C ** 0.5
MAXNORM = (1.0 - 4e-3) / SQRT_C

NS = 16
EB = 80
EPT = E // NS
SUP = EPT // EB
NB = 3
NOUT = (SUP // NB) * NB
RQ = 624


def _make_spmm():
    mesh = plsc.VectorSubcoreMesh(
        core_axis_name="c", subcore_axis_name="s", num_cores=2)

    @functools.partial(
        pl.kernel,
        out_type=jax.ShapeDtypeStruct((2 * N, DH), jnp.float32),
        mesh=mesh,
        scratch_types=[
            pltpu.VMEM((EPT,), jnp.int32),
            pltpu.VMEM((NB, EB, DH), jnp.float32),
            pltpu.VMEM((NB, EB), jnp.int32),
            pltpu.VMEM((NB, EB), jnp.float32),
            pltpu.VMEM_SHARED((N, DH), jnp.float32),
            pltpu.SemaphoreType.DMA((NB,)),
        ],
    )
    def spmm(x_hbm, col_hbm, row_hbm, w_hbm, z_hbm, out_hbm,
             cols_v, rows3_v, rids_v, wslt_v, ysp, gsem):
        c = lax.axis_index("c")
        s = lax.axis_index("s")
        cN = c * N
        hb0 = s * EPT

        pltpu.sync_copy(col_hbm.at[pl.ds(c * E + hb0, EPT)], cols_v)

        pltpu.sync_copy(z_hbm, ysp.at[pl.ds(s * RQ, RQ)])

        @pl.when(s == NS - 1)
        def _():
            pltpu.sync_copy(z_hbm.at[pl.ds(0, 16)],
                            ysp.at[pl.ds(NS * RQ, 16)])

        def fire(k, b):
            base = k * EB
            pltpu.async_copy(row_hbm.at[pl.ds(hb0 + base, EB)],
                             rids_v.at[b], gsem.at[b])
            pltpu.async_copy(w_hbm.at[pl.ds(hb0 + base, EB)],
                             wslt_v.at[b], gsem.at[b])
            pltpu.async_copy(x_hbm.at[cols_v.at[pl.ds(base, EB)]],
                             rows3_v.at[b], gsem.at[b])

        def drain(k, b):
            base = k * EB
            pltpu.make_async_copy(row_hbm.at[pl.ds(hb0 + base, EB)],
                                  rids_v.at[b], gsem.at[b]).wait()
            pltpu.make_async_copy(w_hbm.at[pl.ds(hb0 + base, EB)],
                                  wslt_v.at[b], gsem.at[b]).wait()
            pltpu.make_async_copy(x_hbm.at[cols_v.at[pl.ds(base, EB)]],
                                  rows3_v.at[b], gsem.at[b]).wait()

        def process(k, b, refire):
            drain(k, b)

            def scale_body(gr, _):
                wg = wslt_v[b, pl.ds(gr * 16, 16)]
                for j in range(16):
                    we = wg[j]
                    e = gr * 16 + j
                    for q in range(DH // 16):
                        sl = pl.ds(q * 16, 16)
                        rows3_v[b, e, sl] = rows3_v[b, e, sl] * we
                return 0
            lax.fori_loop(0, EB // 16, scale_body, 0)

            pltpu.sync_copy(rows3_v.at[b], ysp.at[rids_v.at[b]], add=True)

            if refire:
                @pl.when(k + NB < SUP)
                def _():
                    fire(k + NB, b)

        for b in range(NB):
            fire(b, b)

        plsc.subcore_barrier()

        def outer(g, _):
            for b in range(NB):
                process(g * NB + b, b, True)
            return 0

        lax.fori_loop(0, NOUT // NB, outer, 0)
        for k in range(NOUT, SUP):
            process(k, k % NB, False)
        plsc.subcore_barrier()

        pltpu.sync_copy(ysp.at[pl.ds(s * RQ, RQ)],
                        out_hbm.at[pl.ds(cN + s * RQ, RQ)])

        @pl.when(s == NS - 1)
        def _():
            pltpu.sync_copy(ysp.at[pl.ds(NS * RQ, 16)],
                            out_hbm.at[pl.ds(cN + NS * RQ, 16)])

    return spmm


_spmm_cache = []


def _get_spmm():
    if not _spmm_cache:
        _spmm_cache.append(_make_spmm())
    return _spmm_cache[0]


BR = 1000
NBR = N // BR


def _rownorm(x):
    return jnp.sqrt(jnp.clip(jnp.sum(x * x, axis=-1, keepdims=True),
                             MIN_NORM * MIN_NORM, None))


def _clipnorm(n):
    return jnp.clip(n, MIN_NORM, None)


def _s1_body(y0_ref, y1_ref, w_ref, bias_ref, h_ref, acc_ref):
    i = pl.program_id(0)
    w = w_ref[...]
    h = lax.dot_general(y0_ref[...], w[:, :DH], (((1,), (1,)), ((), ())),
                        preferred_element_type=jnp.float32)
    h = h + lax.dot_general(y1_ref[...], w[:, DH:], (((1,), (1,)), ((), ())),
                            preferred_element_type=jnp.float32)

    un = _clipnorm(_rownorm(h))
    e = jnp.tanh(SQRT_C * un) * h / (SQRT_C * un)
    ne = _clipnorm(_rownorm(e))
    e = jnp.where(ne > MAXNORM, e / ne * MAXNORM, e)

    b = bias_ref[0, 0]
    bn = jnp.clip(jnp.abs(b), MIN_NORM, None)
    eb = jnp.tanh(SQRT_C * bn) * b / (SQRT_C * bn)
    nb = jnp.clip(jnp.abs(eb), MIN_NORM, None)
    vb = jnp.where(nb > MAXNORM, eb / nb * MAXNORM, eb)

    x2 = jnp.sum(e * e, axis=-1, keepdims=True)
    y2 = vb * vb
    xy = vb * jnp.sum(e, axis=-1, keepdims=True)
    num = (1.0 + 2.0 * C * xy + C * y2) * e + (1.0 - C * x2) * vb
    den = 1.0 + 2.0 * C * xy + C * C * x2 * y2
    m = num / jnp.clip(den, MIN_NORM, None)

    nm = _clipnorm(_rownorm(m))
    r = jnp.where(nm > MAXNORM, m / nm * MAXNORM, m)
    pn = _clipnorm(_rownorm(r))
    sarg = jnp.clip(SQRT_C * pn, -1.0 + 1e-7, 1.0 - 1e-7)
    atanh = 0.5 * jnp.log((1.0 + sarg) / (1.0 - sarg))
    hl = atanh * r / (SQRT_C * pn)

    h_ref[...] = hl

    @pl.when(i == 0)
    def _():
        acc_ref[...] = jnp.zeros_like(acc_ref)

    ssum = jnp.sum(hl, axis=0, keepdims=True)
    ssq = jnp.sum(hl * hl, axis=0, keepdims=True)
    upd = jnp.concatenate(
        [ssum, ssq, jnp.zeros((6, D), jnp.float32)], axis=0)
    acc_ref[...] = acc_ref[...] + upd


def _stage1(y01, W, bias2d):
    return pl.pallas_call(
        _s1_body,
        grid=(NBR,),
        in_specs=[
            pl.BlockSpec((BR, DH), lambda i: (i, 0)),
            pl.BlockSpec((BR, DH), lambda i: (NBR + i, 0)),
            pl.BlockSpec((D, D), lambda i: (0, 0)),
            pl.BlockSpec((1, 1), lambda i: (0, 0)),
        ],
        out_specs=[
            pl.BlockSpec((BR, D), lambda i: (i, 0)),
            pl.BlockSpec((8, D), lambda i: (0, 0)),
        ],
        out_shape=[
            jax.ShapeDtypeStruct((N, D), jnp.float32),
            jax.ShapeDtypeStruct((8, D), jnp.float32),
        ],
        compiler_params=pltpu.CompilerParams(
            dimension_semantics=("arbitrary",)),
    )(y01, y01, W, bias2d)


def _s3_body(h_ref, acc_ref, gamma_ref, beta_ref, out_ref):
    h = h_ref[...]
    mean = acc_ref[0:1, :] * (1.0 / N)
    ex2 = acc_ref[1:2, :] * (1.0 / N)
    var = ex2 - mean * mean
    xn = (h - mean) / jnp.sqrt(var + 1e-5) * gamma_ref[...] + beta_ref[...]
    out_ref[...] = h + jnp.maximum(xn, 0.0)


def _stage3(h, acc, gamma2d, beta2d):
    return pl.pallas_call(
        _s3_body,
        grid=(NBR,),
        in_specs=[
            pl.BlockSpec((BR, D), lambda i: (i, 0)),
            pl.BlockSpec((8, D), lambda i: (0, 0)),
            pl.BlockSpec((1, D), lambda i: (0, 0)),
            pl.BlockSpec((1, D), lambda i: (0, 0)),
        ],
        out_specs=pl.BlockSpec((BR, D), lambda i: (i, 0)),
        out_shape=jax.ShapeDtypeStruct((N, D), jnp.float32),
        compiler_params=pltpu.CompilerParams(
            dimension_semantics=("arbitrary",)),
    )(h, acc, gamma2d, beta2d)


def kernel(x, edge_index, edge_weight, W, bias, gamma, beta):
    row = edge_index[0]
    col = edge_index[1]
    x01 = jnp.concatenate([x[:, :DH], x[:, DH:]], axis=0)
    cols2 = jnp.concatenate([col, col + N])
    zrows = jnp.zeros((RQ, DH), jnp.float32)
    y01 = _get_spmm()(x01, cols2, row, edge_weight, zrows)
    h, acc = _stage1(y01, W, bias.reshape(1, 1))
    out = _stage3(h, acc, gamma.reshape(1, D), beta.reshape(1, D))
    return out
